# baseline (speedup 1.0000x reference)
import functools

import jax
import jax.numpy as jnp
from jax import lax
from jax.experimental import pallas as pl
from jax.experimental.pallas import tpu as pltpu
from jax.experimental.pallas import tpu_sc as plsc

NC = 2
NS = 16
NW = NC * NS
L = 16

B = 16384
D = 64
N_TBL = 1000000
NCOLS = (N_TBL + 127) // 128
COLS_PW = (NCOLS + NS - 1) // NS
SENT = 600
BPW = B // NW

_mesh = plsc.VectorSubcoreMesh(core_axis_name="c", subcore_axis_name="s")


def _lane_sum_fns():
    lanes = lax.iota(jnp.int32, L)
    perms = [jnp.bitwise_xor(lanes, sh)[:, None] for sh in (8, 4, 2, 1)]
    gdn = lax.GatherDimensionNumbers(
        offset_dims=(), collapsed_slice_dims=(0,), start_index_map=(0,))

    def dyn_gather(x, idx2d):
        return lax.gather(x, idx2d, gdn, (1,),
                          mode=lax.GatherScatterMode.PROMISE_IN_BOUNDS)

    def lane_sum(p):
        for perm in perms:
            p = p + dyn_gather(p, perm)
        return p

    return lanes, dyn_gather, lane_sum


@functools.partial(
    pl.kernel,
    out_type=(jax.ShapeDtypeStruct((B * D,), jnp.float32),
              jax.ShapeDtypeStruct((B * D,), jnp.float32)),
    mesh=_mesh,
    scratch_types=[
        pltpu.VMEM((B + L,), jnp.int32),
        pltpu.VMEM((B + 32,), jnp.int32),
        pltpu.VMEM((B + 32,), jnp.int32),
        pltpu.VMEM((B + 32,), jnp.int32),
        pltpu.VMEM((512 + L,), jnp.int32),
        pltpu.VMEM((512 + L,), jnp.int32),
        pltpu.VMEM((560,), jnp.int32),
        [pltpu.VMEM((D, 128), jnp.float32)] * 4,
        pltpu.VMEM((16, D), jnp.float32),
        [pltpu.SemaphoreType.DMA] * 4,
        [pltpu.SemaphoreType.DMA] * 16,
    ],
)
def _gather_sc(uidx_hbm, iidx_hbm, ut_hbm, vt_hbm, u_out, v_out,
               xidx, hitidx, hitpos, nxt, head, hcnt, collist, banks,
               stage2d, bsems, ssems):
    c = lax.axis_index("c")
    s = lax.axis_index("s")
    lanes, dyn_gather, _ = _lane_sum_fns()
    zeros = jnp.zeros((L,), jnp.float32)

    def rd(buf, i):
        return buf[pl.ds(i, L)][0]

    def wr(buf, i, val):
        cur = buf[pl.ds(i, L)]
        buf[pl.ds(i, L)] = jnp.where(lanes == 0, val, cur)

    def table_phase(x_hbm, tbl_hbm, dst_hbm):
        pltpu.sync_copy(x_hbm, xidx.at[pl.ds(0, B)])
        lo = s * COLS_PW
        hi = jnp.minimum(lo + COLS_PW, NCOLS)

        neg1 = jnp.full((L,), -1, jnp.int32)

        def zf(i, carry):
            head[pl.ds(i * L, L)] = neg1
            hcnt[pl.ds(i * L, L)] = jnp.zeros((L,), jnp.int32)
            return carry
        lax.fori_loop(0, (512 + L) // L, zf, 0)

        def scan_a(h, cnt):
            vec = xidx[pl.ds(h * L, L)]
            tc = lax.shift_right_logical(vec, 7)
            cnt2 = cnt
            for k in range(L):
                idxk = vec[k]
                tck = tc[k]
                inrk = (tck >= lo) & (tck < hi)

                @pl.when(inrk)
                def _(idxk=idxk, tck=tck, cnt2=cnt2, k=k):
                    wr(hitidx, cnt2, idxk)
                    wr(hitpos, cnt2, h * L + k)
                    rel = tck - lo
                    wr(nxt, cnt2, rd(head, rel))
                    wr(head, rel, cnt2)
                    wr(hcnt, rel, rd(hcnt, rel) + 1)
                cnt2 = cnt2 + jnp.where(inrk, 1, 0)
            return cnt2

        cnt = lax.fori_loop(0, B // L, scan_a, 0)

        def scan_f(f, ncol):
            vec = head[pl.ds(f * L, L)]
            ncol2 = ncol
            for k in range(L):
                hk = vec[k]

                @pl.when(hk >= 0)
                def _(ncol2=ncol2, f=f, k=k):
                    wr(collist, ncol2, f * L + k)
                ncol2 = ncol2 + jnp.where(hk >= 0, 1, 0)
            return ncol2
        ncol = lax.fori_loop(0, 512 // L, scan_f, 0)

        collist[pl.ds(ncol, L)] = jnp.full((L,), SENT, jnp.int32)
        total = ((ncol + 3) // 4) * 4
        nhit_v = (cnt + L - 1) // L

        def col_rel(bi):
            return collist[pl.ds(bi, L)][0]

        def issue(bi, j):
            cfetch = jnp.minimum(col_rel(bi) + lo, NCOLS - 1)
            off = pl.multiple_of(cfetch * 128, 128)
            pltpu.async_copy(
                tbl_hbm.at[:, pl.ds(off, 128)], banks[j], bsems[j])

        for i in range(4):
            @pl.when(i < total)
            def _(i=i):
                issue(i, i)

        def quad_body(qq, g):
            for j in range(4):
                bi = qq * 4 + j
                pltpu.make_async_copy(
                    tbl_hbm.at[:, pl.ds(0, 128)], banks[j], bsems[j]).wait()
                cabs = col_rel(bi) + lo
                cbc = jnp.full((L,), cabs, jnp.int32)

                crel = col_rel(bi)

                def walk(t, st, j=j):
                    ptr, g = st
                    idxv = rd(hitidx, ptr)
                    posv = rd(hitpos, ptr)
                    m = jnp.bitwise_and(idxv, 127)
                    msl = jnp.bitwise_and(m, ~15)
                    mlv = jnp.full((L,), jnp.bitwise_and(m, 15),
                                   jnp.int32)[:, None]
                    rows = []
                    for a in range(4):
                        ua = zeros
                        for k in range(L):
                            x = banks[j][a * L + k, pl.ds(msl, L)]
                            y = dyn_gather(x, mlv)
                            ua = jnp.where(lanes == k, y, ua)
                        rows.append(ua)
                    gp = jnp.bitwise_and(g, 15)
                    for par in range(16):
                        @pl.when((gp == par) & (g >= 16))
                        def _(par=par):
                            pltpu.make_async_copy(
                                dst_hbm.at[pl.ds(0, D)], stage2d.at[par],
                                ssems[par]).wait()
                    for a in range(4):
                        stage2d[gp, pl.ds(a * L, L)] = rows[a]
                    for par in range(16):
                        @pl.when(gp == par)
                        def _(par=par, posv=posv):
                            pltpu.async_copy(
                                stage2d.at[par],
                                dst_hbm.at[pl.ds(posv * D, D)],
                                ssems[par])
                    return (rd(nxt, ptr), g + 1)

                safe = crel < COLS_PW
                hd = jnp.where(safe, rd(head, jnp.minimum(crel, 511)), -1)
                nhc = jnp.where(safe, rd(hcnt, jnp.minimum(crel, 511)), 0)
                _, g = lax.fori_loop(0, nhc, walk, (hd, g))

                rn = bi + 4

                @pl.when(rn < total)
                def _(rn=rn, j=j):
                    issue(rn, j)
            return g

        g = lax.fori_loop(0, total // 4, quad_body, 0)

        for par in range(16):
            @pl.when(g >= par + 1)
            def _(par=par):
                pltpu.make_async_copy(
                    dst_hbm.at[pl.ds(0, D)], stage2d.at[par], ssems[par]).wait()

    @pl.when(c == 0)
    def _():
        table_phase(uidx_hbm, ut_hbm, u_out)

    @pl.when(c == 1)
    def _():
        table_phase(iidx_hbm, vt_hbm, v_out)


@functools.partial(
    pl.kernel,
    out_type=jax.ShapeDtypeStruct((B,), jnp.float32),
    mesh=_mesh,
    scratch_types=[
        pltpu.VMEM((BPW * D,), jnp.float32),
        pltpu.VMEM((BPW * D,), jnp.float32),
        pltpu.VMEM((BPW,), jnp.float32),
        pltpu.VMEM((80,), jnp.float32),
    ],
)
def _dot_sc(u_hbm, v_hbm, wb_hbm, out_hbm, uv, vv, outv, wbv):
    wid = lax.axis_index("s") * NC + lax.axis_index("c")
    base = wid * BPW
    pltpu.sync_copy(u_hbm.at[pl.ds(base * D, BPW * D)], uv)
    pltpu.sync_copy(v_hbm.at[pl.ds(base * D, BPW * D)], vv)
    pltpu.sync_copy(wb_hbm, wbv)
    w = [wbv[pl.ds(a * L, L)] for a in range(4)]
    bias = wbv[pl.ds(D, L)][0]
    lanes, _, lane_sum = _lane_sum_fns()
    zeros = jnp.zeros((L,), jnp.float32)

    def group_body(gb, carry):
        r0 = gb * L
        res = zeros
        for k in range(L):
            o = (r0 + k) * D
            p = zeros
            for a in range(4):
                p = p + (uv[pl.ds(o + a * L, L)] * vv[pl.ds(o + a * L, L)]
                         * w[a])
            res = jnp.where(lanes == k, lane_sum(p), res)
        outv[pl.ds(r0, L)] = res + bias
        return carry

    lax.fori_loop(0, BPW // L, group_body, 0)
    pltpu.sync_copy(outv, out_hbm.at[pl.ds(base, BPW)])


def kernel(user_idx, item_idx, U, V, W, b):
    ui = user_idx.astype(jnp.int32)
    ii = item_idx.astype(jnp.int32)
    wb = jnp.concatenate(
        [W.reshape(-1), b.reshape(-1), jnp.zeros((80 - D - 1,), jnp.float32)])
    u_rows, v_rows = _gather_sc(ui, ii, U.T, V.T)
    out = _dot_sc(u_rows, v_rows, wb)
    return out.reshape(B, 1)


# speedup vs baseline: 1.7568x; 1.7568x over previous
import functools

import jax
import jax.numpy as jnp
from jax import lax
from jax.experimental import pallas as pl
from jax.experimental.pallas import tpu as pltpu
from jax.experimental.pallas import tpu_sc as plsc

NC = 2
NS = 16
NW = NC * NS
L = 16

B = 16384
D = 64
N_TBL = 1000000
NCOLS = (N_TBL + 127) // 128
COLS_PW = (NCOLS + NS - 1) // NS
SENT = 600
BPW = B // NW

_mesh = plsc.VectorSubcoreMesh(core_axis_name="c", subcore_axis_name="s")


def _lane_sum_fns():
    lanes = lax.iota(jnp.int32, L)
    perms = [jnp.bitwise_xor(lanes, sh)[:, None] for sh in (8, 4, 2, 1)]
    gdn = lax.GatherDimensionNumbers(
        offset_dims=(), collapsed_slice_dims=(0,), start_index_map=(0,))

    def dyn_gather(x, idx2d):
        return lax.gather(x, idx2d, gdn, (1,),
                          mode=lax.GatherScatterMode.PROMISE_IN_BOUNDS)

    def lane_sum(p):
        for perm in perms:
            p = p + dyn_gather(p, perm)
        return p

    return lanes, dyn_gather, lane_sum


@functools.partial(
    pl.kernel,
    out_type=(jax.ShapeDtypeStruct((B * D,), jnp.float32),
              jax.ShapeDtypeStruct((B * D,), jnp.float32)),
    mesh=_mesh,
    scratch_types=[
        pltpu.VMEM((B + L,), jnp.int32),
        pltpu.VMEM((B + 32,), jnp.int32),
        pltpu.VMEM((B + 32,), jnp.int32),
        pltpu.VMEM((B + 32,), jnp.int32),
        pltpu.VMEM((512 + L,), jnp.int32),
        pltpu.VMEM((512 + L,), jnp.int32),
        pltpu.VMEM((560,), jnp.int32),
        [pltpu.VMEM((D, 128), jnp.float32)] * 4,
        pltpu.VMEM((4, D), jnp.float32),
        [pltpu.SemaphoreType.DMA] * 4,
        [pltpu.SemaphoreType.DMA] * 4,
    ],
)
def _gather_sc(uidx_hbm, iidx_hbm, ut_hbm, vt_hbm, u_out, v_out,
               xidx, hitidx, hitpos, nxt, head, hcnt, collist, banks,
               stage2d, bsems, ssems):
    c = lax.axis_index("c")
    s = lax.axis_index("s")
    lanes, dyn_gather, _ = _lane_sum_fns()
    zeros = jnp.zeros((L,), jnp.float32)

    def rd(buf, i):
        return buf[pl.ds(i, L)][0]

    def wr(buf, i, val):
        cur = buf[pl.ds(i, L)]
        buf[pl.ds(i, L)] = jnp.where(lanes == 0, val, cur)

    def table_phase(x_hbm, tbl_hbm, dst_hbm):
        pltpu.sync_copy(x_hbm, xidx.at[pl.ds(0, B)])
        lo = s * COLS_PW
        hi = jnp.minimum(lo + COLS_PW, NCOLS)

        neg1 = jnp.full((L,), -1, jnp.int32)

        def zf(i, carry):
            head[pl.ds(i * L, L)] = neg1
            hcnt[pl.ds(i * L, L)] = jnp.zeros((L,), jnp.int32)
            return carry
        lax.fori_loop(0, (512 + L) // L, zf, 0)

        def scan_a(h, cnt):
            vec = xidx[pl.ds(h * L, L)]
            tc = lax.shift_right_logical(vec, 7)
            cnt2 = cnt
            for k in range(L):
                idxk = vec[k]
                tck = tc[k]
                inrk = (tck >= lo) & (tck < hi)

                @pl.when(inrk)
                def _(idxk=idxk, tck=tck, cnt2=cnt2, k=k):
                    wr(hitidx, cnt2, idxk)
                    wr(hitpos, cnt2, h * L + k)
                    rel = tck - lo
                    wr(nxt, cnt2, rd(head, rel))
                    wr(head, rel, cnt2)
                    wr(hcnt, rel, rd(hcnt, rel) + 1)
                cnt2 = cnt2 + jnp.where(inrk, 1, 0)
            return cnt2

        cnt = lax.fori_loop(0, B // L, scan_a, 0)

        def scan_f(f, ncol):
            vec = head[pl.ds(f * L, L)]
            ncol2 = ncol
            for k in range(L):
                hk = vec[k]

                @pl.when(hk >= 0)
                def _(ncol2=ncol2, f=f, k=k):
                    wr(collist, ncol2, f * L + k)
                ncol2 = ncol2 + jnp.where(hk >= 0, 1, 0)
            return ncol2
        ncol = lax.fori_loop(0, 512 // L, scan_f, 0)

        collist[pl.ds(ncol, L)] = jnp.full((L,), SENT, jnp.int32)
        total = ((ncol + 3) // 4) * 4
        nhit_v = (cnt + L - 1) // L

        def col_rel(bi):
            return collist[pl.ds(bi, L)][0]

        def issue(bi, j):
            cfetch = jnp.minimum(col_rel(bi) + lo, NCOLS - 1)
            off = pl.multiple_of(cfetch * 128, 128)
            pltpu.async_copy(
                tbl_hbm.at[:, pl.ds(off, 128)], banks[j], bsems[j])

        for i in range(4):
            @pl.when(i < total)
            def _(i=i):
                issue(i, i)

        def quad_body(qq, g):
            for j in range(4):
                bi = qq * 4 + j
                pltpu.make_async_copy(
                    tbl_hbm.at[:, pl.ds(0, 128)], banks[j], bsems[j]).wait()
                cabs = col_rel(bi) + lo
                cbc = jnp.full((L,), cabs, jnp.int32)

                crel = col_rel(bi)

                def walk(t, st, j=j):
                    ptr, g = st
                    idxv = rd(hitidx, ptr)
                    posv = rd(hitpos, ptr)
                    m = jnp.bitwise_and(idxv, 127)
                    msl = jnp.bitwise_and(m, ~15)
                    mlv = jnp.full((L,), jnp.bitwise_and(m, 15),
                                   jnp.int32)[:, None]
                    rows = []
                    for a in range(4):
                        ua = zeros
                        for k in range(L):
                            x = banks[j][a * L + k, pl.ds(msl, L)]
                            y = dyn_gather(x, mlv)
                            ua = jnp.where(lanes == k, y, ua)
                        rows.append(ua)
                    gp = jnp.bitwise_and(g, 3)
                    for par in range(4):
                        @pl.when((gp == par) & (g >= 4))
                        def _(par=par):
                            pltpu.make_async_copy(
                                dst_hbm.at[pl.ds(0, D)], stage2d.at[par],
                                ssems[par]).wait()
                    for a in range(4):
                        stage2d[gp, pl.ds(a * L, L)] = rows[a]
                    for par in range(4):
                        @pl.when(gp == par)
                        def _(par=par, posv=posv):
                            pltpu.async_copy(
                                stage2d.at[par],
                                dst_hbm.at[pl.ds(posv * D, D)],
                                ssems[par])
                    return (rd(nxt, ptr), g + 1)

                safe = crel < COLS_PW
                hd = jnp.where(safe, rd(head, jnp.minimum(crel, 511)), -1)
                nhc = jnp.where(safe, rd(hcnt, jnp.minimum(crel, 511)), 0)
                _, g = lax.fori_loop(0, nhc, walk, (hd, g))

                rn = bi + 4

                @pl.when(rn < total)
                def _(rn=rn, j=j):
                    issue(rn, j)
            return g

        g = lax.fori_loop(0, total // 4, quad_body, 0)

        for par in range(4):
            @pl.when(g >= par + 1)
            def _(par=par):
                pltpu.make_async_copy(
                    dst_hbm.at[pl.ds(0, D)], stage2d.at[par], ssems[par]).wait()

    @pl.when(c == 0)
    def _():
        table_phase(uidx_hbm, ut_hbm, u_out)

    @pl.when(c == 1)
    def _():
        table_phase(iidx_hbm, vt_hbm, v_out)


@functools.partial(
    pl.kernel,
    out_type=jax.ShapeDtypeStruct((B,), jnp.float32),
    mesh=_mesh,
    scratch_types=[
        pltpu.VMEM((BPW * D,), jnp.float32),
        pltpu.VMEM((BPW * D,), jnp.float32),
        pltpu.VMEM((BPW,), jnp.float32),
        pltpu.VMEM((80,), jnp.float32),
    ],
)
def _dot_sc(u_hbm, v_hbm, wb_hbm, out_hbm, uv, vv, outv, wbv):
    wid = lax.axis_index("s") * NC + lax.axis_index("c")
    base = wid * BPW
    pltpu.sync_copy(u_hbm.at[pl.ds(base * D, BPW * D)], uv)
    pltpu.sync_copy(v_hbm.at[pl.ds(base * D, BPW * D)], vv)
    pltpu.sync_copy(wb_hbm, wbv)
    w = [wbv[pl.ds(a * L, L)] for a in range(4)]
    bias = wbv[pl.ds(D, L)][0]
    lanes, _, lane_sum = _lane_sum_fns()
    zeros = jnp.zeros((L,), jnp.float32)

    def group_body(gb, carry):
        r0 = gb * L
        res = zeros
        for k in range(L):
            o = (r0 + k) * D
            p = zeros
            for a in range(4):
                p = p + (uv[pl.ds(o + a * L, L)] * vv[pl.ds(o + a * L, L)]
                         * w[a])
            res = jnp.where(lanes == k, lane_sum(p), res)
        outv[pl.ds(r0, L)] = res + bias
        return carry

    lax.fori_loop(0, BPW // L, group_body, 0)
    pltpu.sync_copy(outv, out_hbm.at[pl.ds(base, BPW)])


def kernel(user_idx, item_idx, U, V, W, b):
    ui = user_idx.astype(jnp.int32)
    ii = item_idx.astype(jnp.int32)
    wb = jnp.concatenate(
        [W.reshape(-1), b.reshape(-1), jnp.zeros((80 - D - 1,), jnp.float32)])
    u_rows, v_rows = _gather_sc(ui, ii, U.T, V.T)
    out = _dot_sc(u_rows, v_rows, wb)
    return out.reshape(B, 1)


# speedup vs baseline: 2.4606x; 1.4006x over previous
import functools

import jax
import jax.numpy as jnp
from jax import lax
from jax.experimental import pallas as pl
from jax.experimental.pallas import tpu as pltpu
from jax.experimental.pallas import tpu_sc as plsc

NC = 2
NS = 16
NW = NC * NS
L = 16

B = 16384
D = 64
BPW = B // NW
NBANK = 4

_mesh = plsc.VectorSubcoreMesh(core_axis_name="c", subcore_axis_name="s")


@functools.partial(
    pl.kernel,
    out_type=jax.ShapeDtypeStruct((B,), jnp.float32),
    mesh=_mesh,
    scratch_types=[
        pltpu.VMEM((BPW + L,), jnp.int32),
        pltpu.VMEM((BPW + L,), jnp.int32),
        [pltpu.VMEM((D, 128), jnp.float32)] * NBANK,
        [pltpu.VMEM((D, 128), jnp.float32)] * NBANK,
        pltpu.VMEM((BPW,), jnp.float32),
        pltpu.VMEM((80,), jnp.float32),
        [pltpu.SemaphoreType.DMA] * NBANK,
        [pltpu.SemaphoreType.DMA] * NBANK,
    ],
)
def _mfnet_sc(uidx_hbm, iidx_hbm, ut_hbm, vt_hbm, wb_hbm, out_hbm,
              uidx_v, iidx_v, ublk, vblk, outv, wbv, sems_u, sems_v):
    wid = lax.axis_index("s") * NC + lax.axis_index("c")
    base = wid * BPW

    pltpu.sync_copy(uidx_hbm.at[pl.ds(base, BPW)], uidx_v.at[pl.ds(0, BPW)])
    pltpu.sync_copy(iidx_hbm.at[pl.ds(base, BPW)], iidx_v.at[pl.ds(0, BPW)])
    pltpu.sync_copy(wb_hbm, wbv)

    w = [wbv[pl.ds(a * L, L)] for a in range(4)]
    bias = wbv[pl.ds(D, L)][0]

    lanes = lax.iota(jnp.int32, L)
    perms = [jnp.bitwise_xor(lanes, sh)[:, None] for sh in (8, 4, 2, 1)]
    gdn = lax.GatherDimensionNumbers(
        offset_dims=(), collapsed_slice_dims=(0,), start_index_map=(0,))

    def lane_sum(p):
        for perm in perms:
            p = p + lax.gather(
                p, perm, gdn, (1,),
                mode=lax.GatherScatterMode.PROMISE_IN_BOUNDS)
        return p

    def row_scalar(idx_ref, r):
        return idx_ref[pl.ds(r, L)][0]

    def issue(idx_ref, tbl, blk, sem, r):
        q = lax.shift_right_logical(row_scalar(idx_ref, r), 7)
        off = pl.multiple_of(q * 128, 128)
        pltpu.async_copy(tbl.at[:, pl.ds(off, 128)], blk, sem)

    for j in range(NBANK):
        issue(uidx_v, ut_hbm, ublk[j], sems_u[j], j)
        issue(iidx_v, vt_hbm, vblk[j], sems_v[j], j)

    zeros = jnp.zeros((L,), jnp.float32)

    def quad_body(qq, res):
        for j in range(NBANK):
            r = qq * NBANK + j
            pltpu.make_async_copy(
                ut_hbm.at[:, pl.ds(0, 128)], ublk[j], sems_u[j]).wait()
            pltpu.make_async_copy(
                vt_hbm.at[:, pl.ds(0, 128)], vblk[j], sems_v[j]).wait()

            cu = jnp.bitwise_and(row_scalar(uidx_v, r), 127)
            ci = jnp.bitwise_and(row_scalar(iidx_v, r), 127)
            msl_u = jnp.bitwise_and(cu, ~15)
            msl_i = jnp.bitwise_and(ci, ~15)
            ml_u = jnp.full((L,), jnp.bitwise_and(cu, 15), jnp.int32)[:, None]
            ml_i = jnp.full((L,), jnp.bitwise_and(ci, 15), jnp.int32)[:, None]
            p = zeros
            for a in range(4):
                ua = zeros
                va = zeros
                for k in range(L):
                    xu = ublk[j][a * L + k, pl.ds(msl_u, L)]
                    yu = lax.gather(xu, ml_u, gdn, (1,),
                                    mode=lax.GatherScatterMode.PROMISE_IN_BOUNDS)
                    ua = jnp.where(lanes == k, yu, ua)
                    xv = vblk[j][a * L + k, pl.ds(msl_i, L)]
                    yv = lax.gather(xv, ml_i, gdn, (1,),
                                    mode=lax.GatherScatterMode.PROMISE_IN_BOUNDS)
                    va = jnp.where(lanes == k, yv, va)
                p = p + ua * va * w[a]
            res = jnp.where(lanes == jnp.bitwise_and(r, 15), lane_sum(p), res)

            rn = r + NBANK
            @pl.when(rn < BPW)
            def _():
                issue(uidx_v, ut_hbm, ublk[j], sems_u[j], rn)
                issue(iidx_v, vt_hbm, vblk[j], sems_v[j], rn)

            if j == NBANK - 1:
                store = jnp.bitwise_and(r, 15) == 15

                @pl.when(store)
                def _():
                    outv[pl.ds(r - 15, L)] = res + bias
                res = jnp.where(store, zeros, res)
        return res

    lax.fori_loop(0, BPW // NBANK, quad_body, zeros)

    pltpu.sync_copy(outv, out_hbm.at[pl.ds(base, BPW)])


def kernel(user_idx, item_idx, U, V, W, b):
    ui = user_idx.astype(jnp.int32)
    ii = item_idx.astype(jnp.int32)
    wb = jnp.concatenate(
        [W.reshape(-1), b.reshape(-1), jnp.zeros((80 - D - 1,), jnp.float32)])
    out = _mfnet_sc(ui, ii, U.T, V.T, wb)
    return out.reshape(B, 1)
